# Initial kernel scaffold; baseline (speedup 1.0000x reference)
import jax
import jax.numpy as jnp
from jax.experimental import pallas as pl


def kernel(embeddings, positive_pairs, negative_pairs):
    raise NotImplementedError("write your pallas kernel here")


# speedup vs baseline: 2.7217x; 2.7217x over previous
import jax
import jax.numpy as jnp
from jax.experimental import pallas as pl
from jax.experimental.pallas import tpu as pltpu

_MARGIN = 1.0
_LAMBDA = 1.0
_TILE_PAIRS = 256


def _normalize_body(x_ref, o_ref):
    x = x_ref[...]
    nsq = jnp.sum(x * x, axis=1, keepdims=True)
    o_ref[...] = x * jax.lax.rsqrt(jnp.maximum(nsq, 1e-16))


def _pair_loss_body(idx_hbm, tab_hbm, out_ref,
                    tab_vmem, prod_vmem, acc_vmem, idx_smem,
                    idx_sem, tab_sem, *,
                    tile_pairs, num_inner, pos_tiles, w_pos, w_neg):
    o = pl.program_id(0)
    i = pl.program_id(1)
    t = 2 * i + o
    slot = jax.lax.rem(i, 2)

    @pl.when(i == 0)
    def _prologue():
        pltpu.make_async_copy(tab_hbm, tab_vmem, tab_sem).start()
        pltpu.make_async_copy(idx_hbm.at[t], idx_smem.at[slot],
                              idx_sem.at[slot]).start()
        pltpu.make_async_copy(tab_hbm, tab_vmem, tab_sem).wait()
        acc_vmem[...] = jnp.zeros_like(acc_vmem)

    @pl.when(i + 1 < num_inner)
    def _prefetch_next():
        t_next = 2 * (i + 1) + o
        slot_next = jax.lax.rem(i + 1, 2)
        pltpu.make_async_copy(idx_hbm.at[t_next], idx_smem.at[slot_next],
                              idx_sem.at[slot_next]).start()

    pltpu.make_async_copy(idx_hbm.at[t], idx_smem.at[slot],
                          idx_sem.at[slot]).wait()

    for mi in range(tile_pairs):
        i1 = idx_smem[slot, mi]
        i2 = idx_smem[slot, tile_pairs + mi]
        r1 = tab_vmem[i1, 0, :]
        r2 = tab_vmem[i2, 0, :]
        prod_vmem[mi, :] = r1 * r2

    dots = jnp.sum(prod_vmem[...], axis=1, keepdims=True)
    dist = 1.0 - dots
    hinge = jnp.maximum(_MARGIN - dist, 0.0)
    is_neg = t >= pos_tiles
    contrib = jnp.where(is_neg, w_neg * hinge, w_pos * dist)
    acc_vmem[...] += contrib

    @pl.when(i == num_inner - 1)
    def _finalize():
        out_ref[...] = jnp.zeros((1, 1, 128), jnp.float32) \
            + jnp.sum(acc_vmem[...])


def _normalize(embeddings):
    n, d = (int(s) for s in embeddings.shape)
    rows = 5000 if n % 10000 == 0 else 8
    grid_inner = n // (2 * rows)
    assert n % (2 * rows) == 0
    return pl.pallas_call(
        _normalize_body,
        out_shape=jax.ShapeDtypeStruct((n, d), jnp.float32),
        grid=(2, grid_inner),
        in_specs=[pl.BlockSpec((rows, d), lambda o, i: (o * (n // (2 * rows)) + i, 0))],
        out_specs=pl.BlockSpec((rows, d), lambda o, i: (o * (n // (2 * rows)) + i, 0)),
        compiler_params=pltpu.CompilerParams(
            dimension_semantics=("parallel", "arbitrary")),
    )(embeddings)


def kernel(embeddings, positive_pairs, negative_pairs):
    num_nodes, emb_dim = (int(s) for s in embeddings.shape)
    num_pos = int(positive_pairs.shape[0])
    num_neg = int(negative_pairs.shape[0])
    tp = _TILE_PAIRS
    assert num_pos % tp == 0 and num_neg % tp == 0

    pos_tiles = num_pos // tp
    num_tiles = pos_tiles + num_neg // tp
    assert num_tiles % 2 == 0
    num_inner = num_tiles // 2

    unit = _normalize(embeddings).reshape(num_nodes, 1, emb_dim)

    pairs = jnp.concatenate([positive_pairs.astype(jnp.int32),
                             negative_pairs.astype(jnp.int32)], axis=0)
    idx = pairs.reshape(num_tiles, tp, 2).transpose(0, 2, 1) \
               .reshape(num_tiles, 2 * tp)

    partials = pl.pallas_call(
        lambda *refs: _pair_loss_body(
            *refs, tile_pairs=tp, num_inner=num_inner, pos_tiles=pos_tiles,
            w_pos=1.0 / num_pos, w_neg=1.0 / num_neg),
        out_shape=jax.ShapeDtypeStruct((2, 1, 128), jnp.float32),
        grid_spec=pltpu.PrefetchScalarGridSpec(
            num_scalar_prefetch=0,
            grid=(2, num_inner),
            in_specs=[
                pl.BlockSpec(memory_space=pl.ANY),
                pl.BlockSpec(memory_space=pl.ANY),
            ],
            out_specs=pl.BlockSpec((1, 1, 128), lambda o, i: (o, 0, 0)),
            scratch_shapes=[
                pltpu.VMEM((num_nodes, 1, emb_dim), jnp.float32),
                pltpu.VMEM((tp, emb_dim), jnp.float32),
                pltpu.VMEM((tp, 1), jnp.float32),
                pltpu.SMEM((2, 2 * tp), jnp.int32),
                pltpu.SemaphoreType.DMA((2,)),
                pltpu.SemaphoreType.DMA,
            ]),
        compiler_params=pltpu.CompilerParams(
            dimension_semantics=("parallel", "arbitrary"),
            vmem_limit_bytes=64 * 1024 * 1024),
    )(idx, unit)

    return _LAMBDA * jnp.sum(partials[:, 0, 0])
